# baseline (speedup 1.0000x reference)
import functools

import jax
import jax.numpy as jnp
from jax.experimental import pallas as pl
from jax.experimental.pallas import tpu as pltpu


def _router_body(x_ref, w_ref, probs_ref, logits_ref, z_ref, *, inv_n):
    xb = x_ref[...]
    logits = jnp.dot(xb, w_ref[...],
                     preferred_element_type=jnp.float32)

    m = jnp.max(logits, axis=-1, keepdims=True)
    e = jnp.exp(logits - m)
    probs_ref[...] = e / jnp.sum(e, axis=-1, keepdims=True)

    absum = jnp.sum(jnp.abs(xb), axis=-1, keepdims=True)
    masked = jnp.where(absum > 0, logits, 0.0)
    logits_ref[...] = masked

    mm = jnp.max(masked, axis=-1, keepdims=True)
    lse = jnp.log(jnp.sum(jnp.exp(masked - mm), axis=-1, keepdims=True)) + mm
    z_ref[...] = jnp.full_like(z_ref, jnp.sum(lse * lse) * inv_n)


def kernel(x, W):
    b, s, d = x.shape
    n = b * s
    e = W.shape[1]
    xf = x.reshape(n, d)

    blk = 512
    nblk = n // blk
    body = functools.partial(_router_body, inv_n=1.0 / n)
    probs, logits, z = pl.pallas_call(
        body,
        grid=(nblk,),
        in_specs=[
            pl.BlockSpec((blk, d), lambda i: (i, 0)),
            pl.BlockSpec((d, e), lambda i: (0, 0)),
        ],
        out_specs=[
            pl.BlockSpec((blk, e), lambda i: (i, 0)),
            pl.BlockSpec((blk, e), lambda i: (i, 0)),
            pl.BlockSpec((1, 1, 1), lambda i: (i, 0, 0)),
        ],
        out_shape=[
            jax.ShapeDtypeStruct((n, e), jnp.float32),
            jax.ShapeDtypeStruct((n, e), jnp.float32),
            jax.ShapeDtypeStruct((nblk, 1, 1), jnp.float32),
        ],
        compiler_params=pltpu.CompilerParams(
            dimension_semantics=("parallel",),
        ),
    )(xf, W)
    return probs, logits, jnp.sum(z)


# speedup vs baseline: 1.1263x; 1.1263x over previous
import functools

import jax
import jax.numpy as jnp
from jax.experimental import pallas as pl
from jax.experimental.pallas import tpu as pltpu


def _router_body(x_ref, w_ref, probs_ref, logits_ref, z_ref, *, inv_n):
    xb = x_ref[...]
    logits = jnp.dot(xb, w_ref[...],
                     preferred_element_type=jnp.float32)

    m = jnp.max(logits, axis=-1, keepdims=True)
    e = jnp.exp(logits - m)
    probs_ref[...] = e / jnp.sum(e, axis=-1, keepdims=True)

    absum = jnp.sum(jnp.abs(xb), axis=-1, keepdims=True)
    masked = jnp.where(absum > 0, logits, 0.0)
    logits_ref[...] = masked

    mm = jnp.max(masked, axis=-1, keepdims=True)
    lse = jnp.log(jnp.sum(jnp.exp(masked - mm), axis=-1, keepdims=True)) + mm
    z_ref[...] = jnp.full_like(z_ref, jnp.sum(lse * lse) * inv_n)


def kernel(x, W):
    b, s, d = x.shape
    n = b * s
    e = W.shape[1]
    xf = x.reshape(n, d)

    blk = 1024
    nblk = n // blk
    body = functools.partial(_router_body, inv_n=1.0 / n)
    probs, logits, z = pl.pallas_call(
        body,
        grid=(nblk,),
        in_specs=[
            pl.BlockSpec((blk, d), lambda i: (i, 0)),
            pl.BlockSpec((d, e), lambda i: (0, 0)),
        ],
        out_specs=[
            pl.BlockSpec((blk, e), lambda i: (i, 0)),
            pl.BlockSpec((blk, e), lambda i: (i, 0)),
            pl.BlockSpec((1, 1, 1), lambda i: (i, 0, 0)),
        ],
        out_shape=[
            jax.ShapeDtypeStruct((n, e), jnp.float32),
            jax.ShapeDtypeStruct((n, e), jnp.float32),
            jax.ShapeDtypeStruct((nblk, 1, 1), jnp.float32),
        ],
        compiler_params=pltpu.CompilerParams(
            dimension_semantics=("parallel",),
        ),
    )(xf, W)
    return probs, logits, jnp.sum(z)
